# baseline (speedup 1.0000x reference)
import functools

import jax
import jax.numpy as jnp
from jax import lax
from jax.experimental import pallas as pl
from jax.experimental.pallas import tpu as pltpu
from jax.experimental.pallas import tpu_sc as plsc

NC, NS = 2, 16
NW = NC * NS
B, D, V = 16384, 512, 10
VP = 16
CH = 16
NB = 8
NCH = B // (NW * CH)
ROWS_W = B // NW
L = 16

_mesh = plsc.VectorSubcoreMesh(core_axis_name="c", subcore_axis_name="s")


@functools.partial(
    pl.kernel,
    mesh=_mesh,
    out_type=jax.ShapeDtypeStruct((B, D), jnp.float32),
    scratch_types=[
        pltpu.VMEM((ROWS_W,), jnp.int32),
        pltpu.VMEM((NB, CH, D), jnp.float32),
        pltpu.VMEM_SHARED((VP, D), jnp.float32),
        pltpu.SemaphoreType.DMA,
        pltpu.SemaphoreType.DMA,
    ],
)
def _lookup_kernel(ids_hbm, table_hbm, out_hbm, idx_v, bufs, table_sh,
                   sem_g, sem_s):
    sid = lax.axis_index("s")
    wid = sid * NC + lax.axis_index("c")
    base = wid * ROWS_W

    @pl.when(sid == 0)
    def _():
        pltpu.sync_copy(table_hbm, table_sh)

    pltpu.sync_copy(ids_hbm.at[pl.ds(base, ROWS_W)], idx_v)
    plsc.subcore_barrier()

    def chunk_body(j, carry):
        slot = lax.rem(j, NB)
        buf = bufs.at[slot]

        @pl.when(j >= NB)
        def _():
            pltpu.make_async_copy(
                out_hbm.at[pl.ds(base, CH)], buf, sem_s).wait()

        def g_body(g, c):
            vec = idx_v[pl.ds(j * CH + g * L, L)]
            for k in range(L):
                pltpu.async_copy(table_sh.at[vec[k]], buf.at[g * L + k], sem_g)
            return c

        lax.fori_loop(0, CH // L, g_body, 0)
        pltpu.make_async_copy(out_hbm.at[pl.ds(base, CH)], buf, sem_g).wait()
        pltpu.async_copy(buf, out_hbm.at[pl.ds(base + j * CH, CH)], sem_s)
        return carry

    lax.fori_loop(0, NCH, chunk_body, 0)
    for b in range(NB):
        pltpu.make_async_copy(out_hbm.at[pl.ds(base, CH)], bufs.at[b], sem_s).wait()


def kernel(class_ids, table):
    table_p = jnp.zeros((VP, D), table.dtype).at[:V].set(table)
    return _lookup_kernel(class_ids, table_p)


# speedup vs baseline: 1.0467x; 1.0467x over previous
import functools

import jax
import jax.numpy as jnp
from jax import lax
from jax.experimental import pallas as pl
from jax.experimental.pallas import tpu as pltpu
from jax.experimental.pallas import tpu_sc as plsc

NC, NS = 2, 16
NW = NC * NS
B, D, V = 16384, 512, 10
VP = 16
CH = 32
NB = 4
NCH = B // (NW * CH)
ROWS_W = B // NW
L = 16

_mesh = plsc.VectorSubcoreMesh(core_axis_name="c", subcore_axis_name="s")


@functools.partial(
    pl.kernel,
    mesh=_mesh,
    out_type=jax.ShapeDtypeStruct((B, D), jnp.float32),
    scratch_types=[
        pltpu.VMEM((ROWS_W,), jnp.int32),
        pltpu.VMEM((NB, CH, D), jnp.float32),
        pltpu.VMEM_SHARED((VP, D), jnp.float32),
        pltpu.SemaphoreType.DMA,
        pltpu.SemaphoreType.DMA,
    ],
)
def _lookup_kernel(ids_hbm, table_hbm, out_hbm, idx_v, bufs, table_sh,
                   sem_g, sem_s):
    sid = lax.axis_index("s")
    wid = sid * NC + lax.axis_index("c")
    base = wid * ROWS_W

    @pl.when(sid == 0)
    def _():
        pltpu.sync_copy(table_hbm, table_sh)

    pltpu.sync_copy(ids_hbm.at[pl.ds(base, ROWS_W)], idx_v)
    plsc.subcore_barrier()

    def chunk_body(j, carry):
        slot = lax.rem(j, NB)
        buf = bufs.at[slot]

        @pl.when(j >= NB)
        def _():
            pltpu.make_async_copy(
                out_hbm.at[pl.ds(base, CH)], buf, sem_s).wait()

        def g_body(g, c):
            vec = idx_v[pl.ds(j * CH + g * L, L)]
            for k in range(L):
                pltpu.async_copy(table_sh.at[vec[k]], buf.at[g * L + k], sem_g)
            return c

        lax.fori_loop(0, CH // L, g_body, 0)
        pltpu.make_async_copy(out_hbm.at[pl.ds(base, CH)], buf, sem_g).wait()
        pltpu.async_copy(buf, out_hbm.at[pl.ds(base + j * CH, CH)], sem_s)
        return carry

    lax.fori_loop(0, NCH, chunk_body, 0)
    for b in range(NB):
        pltpu.make_async_copy(out_hbm.at[pl.ds(base, CH)], bufs.at[b], sem_s).wait()


def kernel(class_ids, table):
    table_p = jnp.zeros((VP, D), table.dtype).at[:V].set(table)
    return _lookup_kernel(class_ids.astype(jnp.int32), table_p)
